# Initial kernel scaffold; baseline (speedup 1.0000x reference)
import jax
import jax.numpy as jnp
from jax.experimental import pallas as pl


def kernel(x, rbfs, xyz_w_1, xyz_b_1, xyz_w_2, xyz_b_2, xyz_w_3, xyz_b_3, xyz_w_4, xyz_b_4, final_w, final_b, dir_w, dir_b, sigma_w, sigma_b, rgb_w, rgb_b):
    raise NotImplementedError("write your pallas kernel here")


# speedup vs baseline: 8.3320x; 8.3320x over previous
import functools

import jax
import jax.numpy as jnp
from jax.experimental import pallas as pl
from jax.experimental.pallas import tpu as pltpu


def _nerflet_step(scores_ref, x_ref,
                  w1, b1, w2, b2, w3, b3, w4, b4,
                  fw, fb, dw, db, sw, sb, rw, rb,
                  out_ref, i1_s, i2_s, *, cxyz, n):
    e = pl.program_id(0)

    @pl.when(e == 0)
    def _route():
        s = scores_ref[...]
        b_, n_ = s.shape
        sb = jax.lax.bitcast_convert_type(s, jnp.int32)
        key = jnp.where(sb < 0, sb ^ jnp.int32(0x7FFFFFFF), sb)
        lane = jax.lax.broadcasted_iota(jnp.int32, (b_, n_), 1)
        m1 = jnp.max(key, axis=1, keepdims=True)
        i1 = jnp.min(jnp.where(key == m1, lane, jnp.int32(n_)),
                     axis=1, keepdims=True)
        key2 = jnp.where(lane == i1, jnp.int32(-2147483648), key)
        m2 = jnp.max(key2, axis=1, keepdims=True)
        i2 = jnp.min(jnp.where(key2 == m2, lane, jnp.int32(n_)),
                     axis=1, keepdims=True)
        i1_s[...] = i1
        i2_s[...] = i2

    xyz = x_ref[:, :cxyz]
    xdir = x_ref[:, cxyz:]
    f32 = jnp.float32

    h = jax.nn.relu(jnp.dot(xyz, w1[0], preferred_element_type=f32) + b1[0])
    h = jax.nn.relu(jnp.dot(h, w2[0], preferred_element_type=f32) + b2[0])
    h = jax.nn.relu(jnp.dot(h, w3[0], preferred_element_type=f32) + b3[0])
    h = jax.nn.relu(jnp.dot(h, w4[0], preferred_element_type=f32) + b4[0])

    sigma = jax.nn.softplus(
        jnp.dot(h, sw[0], preferred_element_type=f32) + sb[0] - 1.0)
    fin = jnp.dot(h, fw[0], preferred_element_type=f32) + fb[0]
    w_hid = dw[0][: fin.shape[1], :]
    w_dir = dw[0][fin.shape[1]:, :]
    dire = jax.nn.relu(
        jnp.dot(fin, w_hid, preferred_element_type=f32)
        + jnp.dot(xdir, w_dir, preferred_element_type=f32) + db[0])
    rgb = jax.nn.sigmoid(jnp.dot(dire, rw[0], preferred_element_type=f32)
                         + rb[0])

    contrib = jnp.concatenate([rgb, sigma], axis=1)
    out_ref[0] = jnp.where(i1_s[...] == e, contrib, out_ref[0])
    out_ref[1] = jnp.where(i2_s[...] == e, contrib, out_ref[1])


def kernel(x, rbfs, xyz_w_1, xyz_b_1, xyz_w_2, xyz_b_2, xyz_w_3, xyz_b_3,
           xyz_w_4, xyz_b_4, final_w, final_b, dir_w, dir_b, sigma_w,
           sigma_b, rgb_w, rgb_b):
    n, b, _ = rbfs.shape
    cxyz = xyz_w_1.shape[1]
    w = xyz_w_1.shape[2]
    whalf = dir_w.shape[2]

    scores = jnp.transpose(jnp.squeeze(rbfs, -1), (1, 0))

    xyz_b_1, xyz_b_2, xyz_b_3, xyz_b_4, final_b, dir_b, sigma_b, rgb_b = (
        a[:, None, :] for a in (xyz_b_1, xyz_b_2, xyz_b_3, xyz_b_4,
                                final_b, dir_b, sigma_b, rgb_b))

    def wspec(i, o):
        return pl.BlockSpec((1, i, o), lambda e: (e, 0, 0))

    def bspec(o):
        return pl.BlockSpec((1, 1, o), lambda e: (e, 0, 0))

    full2 = lambda arr: pl.BlockSpec(arr.shape, lambda e: (0, 0))

    grid_spec = pltpu.PrefetchScalarGridSpec(
        num_scalar_prefetch=0,
        grid=(n,),
        in_specs=[
            full2(scores),
            full2(x),
            wspec(cxyz, w), bspec(w),
            wspec(w, w), bspec(w),
            wspec(w, w), bspec(w),
            wspec(w, w), bspec(w),
            wspec(w, w), bspec(w),
            wspec(w + x.shape[1] - cxyz, whalf), bspec(whalf),
            wspec(w, 1), bspec(1),
            wspec(whalf, rgb_w.shape[2]), bspec(rgb_w.shape[2]),
        ],
        out_specs=pl.BlockSpec((2, b, 4), lambda e: (0, 0, 0)),
        scratch_shapes=[
            pltpu.VMEM((b, 1), jnp.int32),
            pltpu.VMEM((b, 1), jnp.int32),
        ],
    )

    out = pl.pallas_call(
        functools.partial(_nerflet_step, cxyz=cxyz, n=n),
        grid_spec=grid_spec,
        out_shape=jax.ShapeDtypeStruct((2, b, 4), jnp.float32),
        compiler_params=pltpu.CompilerParams(
            dimension_semantics=("arbitrary",),
        ),
    )(scores, x,
      xyz_w_1, xyz_b_1, xyz_w_2, xyz_b_2, xyz_w_3, xyz_b_3,
      xyz_w_4, xyz_b_4, final_w, final_b, dir_w, dir_b,
      sigma_w, sigma_b, rgb_w, rgb_b)
    return out
